# baseline (speedup 1.0000x reference)
import functools
import math

import jax
import jax.numpy as jnp
from jax.experimental import pallas as pl
from jax.experimental.pallas import tpu as pltpu

S = 64.0
M = 0.35
COS_M = math.cos(M)
SIN_M = math.sin(M)
THRESHOLD = math.cos(math.pi - M)
SHIFT = 64.0

R = 16
NSLOT = 4


def _arc_kernel(t_ref, x_hbm, o_ref, buf, sems, *, n_blk, b_total):
    i = pl.program_id(0)

    def copy(blk, slot):
        return pltpu.make_async_copy(
            x_hbm.at[pl.ds(blk * R, R), :],
            buf.at[slot],
            sems.at[slot],
        )

    @pl.when(i == 0)
    def _():
        for k in range(NSLOT - 1):
            copy(k, k).start()

    slot = jax.lax.rem(i, NSLOT)
    copy(i, slot).wait()

    lane_iota = jax.lax.broadcasted_iota(jnp.int32, (1, 128), 1)
    cts = []
    for r in range(R):
        tv = t_ref[0, 0, r]
        base = pl.multiple_of((tv // 128) * 128, 128)
        lane = tv - base
        chunk = buf[slot, pl.ds(r, 1), pl.ds(base, 128)]
        hit = lane_iota == lane
        cts.append(jnp.sum(jnp.where(hit, chunk, 0.0), axis=1, keepdims=True))
        buf[slot, pl.ds(r, 1), pl.ds(base, 128)] = jnp.where(hit, -1e30, chunk)
    ct = jnp.concatenate(cts, axis=0)[:, 0]

    x = buf[slot]
    e = jnp.exp(S * x - SHIFT)
    s = jnp.sum(e, axis=1)

    sin = jnp.clip(jnp.sqrt(jnp.maximum(1.0 - ct * ct, 0.0)), 0.0, 1.0)
    ctm = jnp.clip(ct * COS_M - sin * SIN_M, -1.0, 1.0)
    phi = ct - M * SIN_M
    g = jnp.where(ct > THRESHOLD, ctm, phi)
    out_t = S * g

    total = s + jnp.exp(out_t - SHIFT)
    li = (SHIFT + jnp.log(total)) - out_t
    contrib = jnp.sum(li) / b_total

    @pl.when(i == 0)
    def _():
        o_ref[...] = jnp.zeros_like(o_ref)

    o_ref[...] += jnp.full((1, 1), contrib, dtype=jnp.float32)

    nxt = i + NSLOT - 1

    @pl.when(nxt < n_blk)
    def _():
        copy(nxt, jax.lax.rem(nxt, NSLOT)).start()


def kernel(cos_theta, target):
    B, C = cos_theta.shape
    n_blk = B // R
    t3 = target.astype(jnp.int32).reshape(n_blk, 1, R)

    out = pl.pallas_call(
        functools.partial(_arc_kernel, n_blk=n_blk, b_total=float(B)),
        grid=(n_blk,),
        in_specs=[
            pl.BlockSpec((1, 1, R), lambda i: (i, 0, 0), memory_space=pltpu.SMEM),
            pl.BlockSpec(memory_space=pl.ANY),
        ],
        out_specs=pl.BlockSpec((1, 1), lambda i: (0, 0)),
        out_shape=jax.ShapeDtypeStruct((1, 1), jnp.float32),
        scratch_shapes=[
            pltpu.VMEM((NSLOT, R, C), jnp.float32),
            pltpu.SemaphoreType.DMA((NSLOT,)),
        ],
    )(t3, cos_theta)
    return out[0, 0]


# speedup vs baseline: 1.0002x; 1.0002x over previous
import functools
import math

import jax
import jax.numpy as jnp
from jax.experimental import pallas as pl
from jax.experimental.pallas import tpu as pltpu

S = 64.0
M = 0.35
COS_M = math.cos(M)
SIN_M = math.sin(M)
THRESHOLD = math.cos(math.pi - M)
SHIFT = 64.0


def _arc_kernel(t_ref, x_ref, o_ref, *, n_rows, b_total):
    i = pl.program_id(0)

    lane_iota = jax.lax.broadcasted_iota(jnp.int32, (1, 128), 1)
    cts = []
    for r in range(n_rows):
        tv = t_ref[0, 0, r]
        base = pl.multiple_of((tv // 128) * 128, 128)
        lane = tv - base
        chunk = x_ref[pl.ds(r, 1), pl.ds(base, 128)]
        hit = lane_iota == lane
        cts.append(jnp.sum(jnp.where(hit, chunk, 0.0), axis=1, keepdims=True))
        x_ref[pl.ds(r, 1), pl.ds(base, 128)] = jnp.where(hit, -1e30, chunk)
    ct = jnp.concatenate(cts, axis=0)[:, 0]

    x = x_ref[...]
    e = jnp.exp(S * x - SHIFT)
    s = jnp.sum(e, axis=1)

    sin = jnp.clip(jnp.sqrt(jnp.maximum(1.0 - ct * ct, 0.0)), 0.0, 1.0)
    ctm = jnp.clip(ct * COS_M - sin * SIN_M, -1.0, 1.0)
    phi = ct - M * SIN_M
    g = jnp.where(ct > THRESHOLD, ctm, phi)
    out_t = S * g

    total = s + jnp.exp(out_t - SHIFT)
    li = (SHIFT + jnp.log(total)) - out_t
    contrib = jnp.sum(li) / b_total

    @pl.when(i == 0)
    def _():
        o_ref[...] = jnp.zeros_like(o_ref)

    o_ref[...] += jnp.full((1, 1), contrib, dtype=jnp.float32)


def kernel(cos_theta, target):
    B, C = cos_theta.shape
    R = 64
    n_blk = B // R
    t3 = target.astype(jnp.int32).reshape(n_blk, 1, R)

    out = pl.pallas_call(
        functools.partial(_arc_kernel, n_rows=R, b_total=float(B)),
        grid=(n_blk,),
        in_specs=[
            pl.BlockSpec((1, 1, R), lambda i: (i, 0, 0), memory_space=pltpu.SMEM),
            pl.BlockSpec((R, C), lambda i: (i, 0)),
        ],
        out_specs=pl.BlockSpec((1, 1), lambda i: (0, 0)),
        out_shape=jax.ShapeDtypeStruct((1, 1), jnp.float32),
    )(t3, cos_theta)
    return out[0, 0]
